# baseline (speedup 1.0000x reference)
import jax
import jax.numpy as jnp
from jax.experimental import pallas as pl
from jax.experimental.pallas import tpu as pltpu

QUEUE_ROWS = 32768
CHUNK = 2048
NBUF = 16


def _fifo_copy(x_ref, q_ref, o_ref, buf, sin, sout):
    shift = 4096
    keep = QUEUE_ROWS - shift
    n_q = keep // CHUNK
    n_chunks = QUEUE_ROWS // CHUNK

    ins = []
    outs = []
    for c in range(n_chunks):
        b = c % NBUF
        if c < n_q:
            src = q_ref.at[pl.ds(shift + c * CHUNK, CHUNK)]
        else:
            src = x_ref.at[pl.ds((c - n_q) * CHUNK, CHUNK)]
        ins.append(pltpu.make_async_copy(src, buf.at[b], sin.at[b]))
        outs.append(pltpu.make_async_copy(
            buf.at[b], o_ref.at[pl.ds(c * CHUNK, CHUNK)], sout.at[b]))

    for c in range(n_chunks):
        ins[c].start()
    for c in range(n_chunks):
        ins[c].wait()
        outs[c].start()
    for c in range(n_chunks):
        outs[c].wait()


def kernel(x, queue):
    return pl.pallas_call(
        _fifo_copy,
        out_shape=jax.ShapeDtypeStruct(queue.shape, queue.dtype),
        in_specs=[
            pl.BlockSpec(memory_space=pl.ANY),
            pl.BlockSpec(memory_space=pl.ANY),
        ],
        out_specs=pl.BlockSpec(memory_space=pl.ANY),
        scratch_shapes=[
            pltpu.VMEM((NBUF, CHUNK, 128), jnp.float32),
            pltpu.SemaphoreType.DMA((NBUF,)),
            pltpu.SemaphoreType.DMA((NBUF,)),
        ],
    )(x, queue)


# speedup vs baseline: 1.0279x; 1.0279x over previous
import jax
import jax.numpy as jnp
from jax.experimental import pallas as pl
from jax.experimental.pallas import tpu as pltpu

QUEUE_ROWS = 32768
SHIFT = 4096
CHUNKS = (
    [(512, False), (1024, False), (2560, False)]
    + [(4096, False)] * 6
    + [(2048, True), (1024, True), (512, True), (512, True)]
)
N_CHUNKS = len(CHUNKS)


def _fifo_copy(x_ref, q_ref, o_ref, buf, sin, sout):
    ins = []
    outs = []
    out_off = 0
    x_off = 0
    for c, (rows, from_x) in enumerate(CHUNKS):
        if from_x:
            src = x_ref.at[pl.ds(x_off, rows)]
            x_off += rows
        else:
            src = q_ref.at[pl.ds(SHIFT + out_off, rows)]
        stage = buf.at[pl.ds(out_off, rows)]
        ins.append(pltpu.make_async_copy(src, stage, sin.at[c]))
        outs.append(pltpu.make_async_copy(
            stage, o_ref.at[pl.ds(out_off, rows)], sout.at[c]))
        out_off += rows

    for c in range(N_CHUNKS):
        ins[c].start()
    for c in range(N_CHUNKS):
        ins[c].wait()
        outs[c].start()
    for c in range(N_CHUNKS):
        outs[c].wait()


def kernel(x, queue):
    return pl.pallas_call(
        _fifo_copy,
        out_shape=jax.ShapeDtypeStruct(queue.shape, queue.dtype),
        in_specs=[
            pl.BlockSpec(memory_space=pl.ANY),
            pl.BlockSpec(memory_space=pl.ANY),
        ],
        out_specs=pl.BlockSpec(memory_space=pl.ANY),
        scratch_shapes=[
            pltpu.VMEM((QUEUE_ROWS, 128), jnp.float32),
            pltpu.SemaphoreType.DMA((N_CHUNKS,)),
            pltpu.SemaphoreType.DMA((N_CHUNKS,)),
        ],
    )(x, queue)
